# initial kernel scaffold (unmeasured)
import jax
import jax.numpy as jnp
from jax import lax
from jax.experimental import pallas as pl
from jax.experimental.pallas import tpu as pltpu

N_DEV = 32


def kernel(x, w_mat):
    k, m_per = x.shape
    _, n = w_mat.shape
    assert k == N_DEV * m_per

    def body(x_ref, w_ref, out_ref, comm_ref, send_sems, recv_sems):
        my = lax.axis_index("i")
        left = lax.rem(my - 1 + N_DEV, N_DEV)
        right = lax.rem(my + 1, N_DEV)

        barrier_sem = pltpu.get_barrier_semaphore()
        for nbr in (left, right):
            pl.semaphore_signal(
                barrier_sem, inc=1,
                device_id=(nbr,), device_id_type=pl.DeviceIdType.MESH,
            )
        pl.semaphore_wait(barrier_sem, 2)

        def partial_chunk(c):
            xs = x_ref[pl.ds(c * m_per, m_per), :]
            return jnp.dot(xs, w_ref[:, :], preferred_element_type=jnp.float32)

        comm_ref[0] = partial_chunk(lax.rem(my - 1 + N_DEV, N_DEV))

        for s in range(N_DEV - 1):
            rdma = pltpu.make_async_remote_copy(
                src_ref=comm_ref.at[s],
                dst_ref=comm_ref.at[s + 1],
                send_sem=send_sems.at[s],
                recv_sem=recv_sems.at[s],
                device_id=(right,),
                device_id_type=pl.DeviceIdType.MESH,
            )
            rdma.start()
            rdma.wait()
            c = lax.rem(my - s - 2 + 2 * N_DEV, N_DEV)
            comm_ref[s + 1] = comm_ref[s + 1] + partial_chunk(c)

        y = comm_ref[N_DEV - 1]
        out_ref[:, :] = y * jax.nn.sigmoid(y)

    return pl.pallas_call(
        body,
        out_shape=jax.ShapeDtypeStruct((m_per, n), jnp.float32),
        in_specs=[
            pl.BlockSpec(memory_space=pltpu.VMEM),
            pl.BlockSpec(memory_space=pltpu.VMEM),
        ],
        out_specs=pl.BlockSpec(memory_space=pltpu.VMEM),
        scratch_shapes=[
            pltpu.VMEM((N_DEV, m_per, n), jnp.float32),
            pltpu.SemaphoreType.DMA((N_DEV - 1,)),
            pltpu.SemaphoreType.DMA((N_DEV - 1,)),
        ],
        compiler_params=pltpu.CompilerParams(collective_id=0),
    )(x, w_mat)


# baseline (device time: 422474 ns/iter reference)
import jax
import jax.numpy as jnp
from jax import lax
from jax.experimental import pallas as pl
from jax.experimental.pallas import tpu as pltpu

N_DEV = 32


def kernel(x, w_mat):
    k, m_per = x.shape
    _, n = w_mat.shape
    assert k == N_DEV * m_per

    def body(x_ref, w_ref, out_ref, comm_ref, *sems):
        send_sems = sems[: N_DEV - 1]
        recv_sems = sems[N_DEV - 1 :]
        my = lax.axis_index("i")
        left = lax.rem(my - 1 + N_DEV, N_DEV)
        right = lax.rem(my + 1, N_DEV)

        barrier_sem = pltpu.get_barrier_semaphore()
        for nbr in (left, right):
            pl.semaphore_signal(
                barrier_sem, inc=1,
                device_id=(nbr,), device_id_type=pl.DeviceIdType.MESH,
            )
        pl.semaphore_wait(barrier_sem, 2)

        def partial_chunk(c):
            xs = x_ref[pl.ds(c * m_per, m_per), :]
            return jnp.dot(xs, w_ref[:, :], preferred_element_type=jnp.float32)

        comm_ref[0] = partial_chunk(lax.rem(my - 1 + N_DEV, N_DEV))

        for s in range(N_DEV - 1):
            rdma = pltpu.make_async_remote_copy(
                src_ref=comm_ref.at[s],
                dst_ref=comm_ref.at[s + 1],
                send_sem=send_sems[s],
                recv_sem=recv_sems[s],
                device_id=(right,),
                device_id_type=pl.DeviceIdType.MESH,
            )
            rdma.start()
            rdma.wait()
            c = lax.rem(my - s - 2 + 2 * N_DEV, N_DEV)
            comm_ref[s + 1] = comm_ref[s + 1] + partial_chunk(c)

        y = comm_ref[N_DEV - 1]
        out_ref[:, :] = y * jax.nn.sigmoid(y)

    return pl.pallas_call(
        body,
        out_shape=jax.ShapeDtypeStruct((m_per, n), jnp.float32),
        in_specs=[
            pl.BlockSpec(memory_space=pltpu.VMEM),
            pl.BlockSpec(memory_space=pltpu.VMEM),
        ],
        out_specs=pl.BlockSpec(memory_space=pltpu.VMEM),
        scratch_shapes=[
            pltpu.VMEM((N_DEV, m_per, n), jnp.float32),
        ]
        + [pltpu.SemaphoreType.DMA] * (2 * (N_DEV - 1)),
        compiler_params=pltpu.CompilerParams(
            collective_id=0,
            vmem_limit_bytes=48 * 1024 * 1024,
        ),
    )(x, w_mat)


# device time: 251662 ns/iter; 1.6787x vs baseline; 1.6787x over previous
import numpy as np

import jax
import jax.numpy as jnp
from jax import lax
from jax.experimental import pallas as pl
from jax.experimental.pallas import tpu as pltpu

N_DEV = 32
H = N_DEV - 1


def _logical_coords():
    order = []
    for z in range(4):
        plane = sorted((x, y, z) for x in range(2) for y in range(4))
        ys = sorted({c[1] for c in plane})
        for yi, y in enumerate(ys):
            row = sorted((c for c in plane if c[1] == y), reverse=bool(yi % 2))
            order.extend(row)
    return order


def _cycle_perm():
    plane = []
    for y in range(4):
        zs = range(4) if y % 2 == 0 else range(3, -1, -1)
        plane.extend((y, z) for z in zs)
    cycle = [(0, y, z) for (y, z) in plane] + [(1, y, z) for (y, z) in reversed(plane)]
    for a, b in zip(cycle, cycle[1:] + cycle[:1]):
        assert sum(abs(u - v) for u, v in zip(a, b)) == 1, (a, b)
    logical = {c: i for i, c in enumerate(_logical_coords())}
    ids = [logical[c] for c in cycle]
    sigma = np.zeros(N_DEV, dtype=np.int32)
    for a, b in zip(ids, ids[1:] + ids[:1]):
        sigma[a] = b
    return sigma


_SIGMA = _cycle_perm()
_POW = np.zeros((N_DEV, N_DEV), dtype=np.int32)
_POW[0] = np.arange(N_DEV, dtype=np.int32)
for _k in range(1, N_DEV):
    _POW[_k] = _SIGMA[_POW[_k - 1]]
assert (np.sort(_POW[1]) == np.arange(N_DEV)).all()
assert (_SIGMA[_POW[N_DEV - 1]] == _POW[0]).all()


def kernel(x, w_mat):
    k, m_per = x.shape
    _, n = w_mat.shape
    nh = n // 2
    assert k == N_DEV * m_per

    pow_tab = jnp.asarray(_POW)

    def body(pow_ref, x_ref, w_ref, out_ref, comm_p, comm_m, *sems):
        send_p = sems[0 * H : 1 * H]
        recv_p = sems[1 * H : 2 * H]
        send_m = sems[2 * H : 3 * H]
        recv_m = sems[3 * H : 4 * H]

        my = lax.axis_index("i")
        nxt = pow_ref[1, my]
        prv = pow_ref[N_DEV - 1, my]

        barrier_sem = pltpu.get_barrier_semaphore()
        for nbr in (nxt, prv):
            pl.semaphore_signal(
                barrier_sem, inc=1,
                device_id=(nbr,), device_id_type=pl.DeviceIdType.MESH,
            )
        pl.semaphore_wait(barrier_sem, 2)

        def partial_chunk(c, lo):
            xs = x_ref[pl.ds(c * m_per, m_per), :]
            return jnp.dot(xs, w_ref[:, lo : lo + nh],
                           preferred_element_type=jnp.float32)

        comm_p[0] = partial_chunk(pow_ref[N_DEV - 1, my], 0)
        comm_m[0] = partial_chunk(pow_ref[1, my], nh)

        for s in range(H):
            rdma_p = pltpu.make_async_remote_copy(
                src_ref=comm_p.at[s], dst_ref=comm_p.at[s + 1],
                send_sem=send_p[s], recv_sem=recv_p[s],
                device_id=(nxt,), device_id_type=pl.DeviceIdType.MESH,
            )
            rdma_m = pltpu.make_async_remote_copy(
                src_ref=comm_m.at[s], dst_ref=comm_m.at[s + 1],
                send_sem=send_m[s], recv_sem=recv_m[s],
                device_id=(prv,), device_id_type=pl.DeviceIdType.MESH,
            )
            rdma_p.start()
            rdma_m.start()
            rdma_p.wait()
            comm_p[s + 1] = comm_p[s + 1] + partial_chunk(
                pow_ref[H - 1 - s, my], 0)
            rdma_m.wait()
            comm_m[s + 1] = comm_m[s + 1] + partial_chunk(
                pow_ref[(s + 2) % N_DEV, my], nh)

        yp = comm_p[H]
        ym = comm_m[H]
        out_ref[:, 0:nh] = yp * jax.nn.sigmoid(yp)
        out_ref[:, nh:n] = ym * jax.nn.sigmoid(ym)

    return pl.pallas_call(
        body,
        out_shape=jax.ShapeDtypeStruct((m_per, n), jnp.float32),
        in_specs=[
            pl.BlockSpec(memory_space=pltpu.SMEM),
            pl.BlockSpec(memory_space=pltpu.VMEM),
            pl.BlockSpec(memory_space=pltpu.VMEM),
        ],
        out_specs=pl.BlockSpec(memory_space=pltpu.VMEM),
        scratch_shapes=[
            pltpu.VMEM((N_DEV, m_per, nh), jnp.float32),
            pltpu.VMEM((N_DEV, m_per, nh), jnp.float32),
        ]
        + [pltpu.SemaphoreType.DMA] * (4 * H),
        compiler_params=pltpu.CompilerParams(
            collective_id=0,
            vmem_limit_bytes=48 * 1024 * 1024,
        ),
    )(pow_tab, x, w_mat)


# device time: 189007 ns/iter; 2.2352x vs baseline; 1.3315x over previous
import numpy as np

import jax
import jax.numpy as jnp
from jax import lax
from jax.experimental import pallas as pl
from jax.experimental.pallas import tpu as pltpu

N_DEV = 32
H = N_DEV - 1
F = 2


def _logical_coords():
    order = []
    for z in range(4):
        plane = sorted((x, y, z) for x in range(2) for y in range(4))
        ys = sorted({c[1] for c in plane})
        for yi, y in enumerate(ys):
            row = sorted((c for c in plane if c[1] == y), reverse=bool(yi % 2))
            order.extend(row)
    return order


def _cycle_perm():
    plane = []
    for y in range(4):
        zs = range(4) if y % 2 == 0 else range(3, -1, -1)
        plane.extend((y, z) for z in zs)
    cycle = [(0, y, z) for (y, z) in plane] + [(1, y, z) for (y, z) in reversed(plane)]
    for a, b in zip(cycle, cycle[1:] + cycle[:1]):
        assert sum(abs(u - v) for u, v in zip(a, b)) == 1, (a, b)
    logical = {c: i for i, c in enumerate(_logical_coords())}
    ids = [logical[c] for c in cycle]
    sigma = np.zeros(N_DEV, dtype=np.int32)
    for a, b in zip(ids, ids[1:] + ids[:1]):
        sigma[a] = b
    return sigma


_SIGMA = _cycle_perm()
_POW = np.zeros((N_DEV, N_DEV), dtype=np.int32)
_POW[0] = np.arange(N_DEV, dtype=np.int32)
for _k in range(1, N_DEV):
    _POW[_k] = _SIGMA[_POW[_k - 1]]


def kernel(x, w_mat):
    k, m_per = x.shape
    _, n = w_mat.shape
    nh = n // 2
    mf = m_per // F
    assert k == N_DEV * m_per

    pow_tab = jnp.asarray(_POW)

    def body(pow_ref, x_ref, w_ref, out_ref, comm_p, comm_m, *sems):
        def sem(d, s, f, r):
            return sems[((d * H + s) * F + f) * 2 + r]

        my = lax.axis_index("i")
        nxt = pow_ref[1, my]
        prv = pow_ref[N_DEV - 1, my]

        barrier_sem = pltpu.get_barrier_semaphore()
        for nbr in (nxt, prv):
            pl.semaphore_signal(
                barrier_sem, inc=1,
                device_id=(nbr,), device_id_type=pl.DeviceIdType.MESH,
            )
        pl.semaphore_wait(barrier_sem, 2)

        def partial_frag(c, f, lo):
            xs = x_ref[pl.ds(c * m_per + f * mf, mf), :]
            return jnp.dot(xs, w_ref[:, lo : lo + nh],
                           preferred_element_type=jnp.float32)

        def mk(dir_idx, comm, s, f, tgt):
            return pltpu.make_async_remote_copy(
                src_ref=comm.at[s, pl.ds(f * mf, mf)],
                dst_ref=comm.at[s + 1, pl.ds(f * mf, mf)],
                send_sem=sem(dir_idx, s, f, 0),
                recv_sem=sem(dir_idx, s, f, 1),
                device_id=(tgt,), device_id_type=pl.DeviceIdType.MESH,
            )

        def cp(s):
            return pow_ref[H - 1 - s, my]

        def cm(s):
            return pow_ref[(s + 2) % N_DEV, my]

        comm_p[0, pl.ds(0, mf)] = partial_frag(pow_ref[N_DEV - 1, my], 0, 0)
        comm_p[0, pl.ds(mf, mf)] = partial_frag(pow_ref[N_DEV - 1, my], 1, 0)
        comm_m[0, pl.ds(0, mf)] = partial_frag(pow_ref[1, my], 0, nh)
        comm_m[0, pl.ds(mf, mf)] = partial_frag(pow_ref[1, my], 1, nh)

        rp = [[mk(0, comm_p, s, f, nxt) for f in range(F)] for s in range(H)]
        rm = [[mk(1, comm_m, s, f, prv) for f in range(F)] for s in range(H)]

        for f in range(F):
            rp[0][f].start()
            rm[0][f].start()

        for s in range(H):
            for f in range(F):
                rp[s][f].wait_recv()
                comm_p[s + 1, pl.ds(f * mf, mf)] = (
                    comm_p[s + 1, pl.ds(f * mf, mf)] + partial_frag(cp(s), f, 0)
                )
                if s + 1 < H:
                    rp[s + 1][f].start()
            for f in range(F):
                rm[s][f].wait_recv()
                comm_m[s + 1, pl.ds(f * mf, mf)] = (
                    comm_m[s + 1, pl.ds(f * mf, mf)] + partial_frag(cm(s), f, nh)
                )
                if s + 1 < H:
                    rm[s + 1][f].start()

        for s in range(H):
            for f in range(F):
                rp[s][f].wait_send()
                rm[s][f].wait_send()

        yp = comm_p[H]
        ym = comm_m[H]
        out_ref[:, 0:nh] = yp * jax.nn.sigmoid(yp)
        out_ref[:, nh:n] = ym * jax.nn.sigmoid(ym)

    return pl.pallas_call(
        body,
        out_shape=jax.ShapeDtypeStruct((m_per, n), jnp.float32),
        in_specs=[
            pl.BlockSpec(memory_space=pltpu.SMEM),
            pl.BlockSpec(memory_space=pltpu.VMEM),
            pl.BlockSpec(memory_space=pltpu.VMEM),
        ],
        out_specs=pl.BlockSpec(memory_space=pltpu.VMEM),
        scratch_shapes=[
            pltpu.VMEM((N_DEV, m_per, nh), jnp.float32),
            pltpu.VMEM((N_DEV, m_per, nh), jnp.float32),
        ]
        + [pltpu.SemaphoreType.DMA] * (2 * H * F * 2),
        compiler_params=pltpu.CompilerParams(
            collective_id=0,
            vmem_limit_bytes=48 * 1024 * 1024,
        ),
    )(pow_tab, x, w_mat)
